# baseline (speedup 1.0000x reference)
import jax
import jax.numpy as jnp
from jax.experimental import pallas as pl
from jax.experimental.pallas import tpu as pltpu

NCLS = 21
ROWS = 256
SUB = 16
NB = 8
NR = 512 // ROWS
EPS = 2.220446049250313e-16


def _dice_body(x_ref, t_ref, out_ref, acc_ref, i16_ref, t16_ref):
    b = pl.program_id(0)
    r = pl.program_id(1)

    @pl.when(jnp.logical_and(b == 0, r == 0))
    def _init():
        acc_ref[...] = jnp.zeros_like(acc_ref)

    for s in range(0, ROWS, SUB):
        best = x_ref[0, 0, s:s + SUB]
        idx = jnp.zeros((SUB, 512), jnp.int32)
        for c in range(1, NCLS):
            xc = x_ref[0, c, s:s + SUB]
            m = xc > best
            best = jnp.where(m, xc, best)
            idx = jnp.where(m, c, idx)
        i16_ref[s // SUB] = idx.astype(jnp.int16)
        t16_ref[s // SUB] = t_ref[0, s:s + SUB].astype(jnp.int16)

    one = jnp.int16(1)
    zero = jnp.int16(0)
    for c in range(NCLS):
        ap = acc_ref[0, c]
        al = acc_ref[1, c]
        ai = acc_ref[2, c]
        for k in range(ROWS // SUB):
            i16 = i16_ref[k]
            t16 = t16_ref[k]
            fp = jnp.where(i16 == c, one, zero)
            fl = jnp.where(t16 == c, one, zero)
            ap = ap + fp
            al = al + fl
            ai = ai + fp * fl
        acc_ref[0, c] = ap
        acc_ref[1, c] = al
        acc_ref[2, c] = ai

    @pl.when(jnp.logical_and(b == NB - 1, r == NR - 1))
    def _fin():
        for c in range(NCLS):
            ai = jnp.sum(acc_ref[2, c].astype(jnp.float32))
            union = (jnp.sum(acc_ref[0, c].astype(jnp.float32))
                     + jnp.sum(acc_ref[1, c].astype(jnp.float32)) - ai)
            iou = ai / (jnp.float32(EPS) + union)
            out_ref[0, c] = 2.0 * iou / (iou + 1.0)
        for c in range(NCLS, 32):
            out_ref[0, c] = 0.0


def kernel(output, target):
    res = pl.pallas_call(
        _dice_body,
        grid=(NB, NR),
        in_specs=[
            pl.BlockSpec((1, NCLS, ROWS, 512), lambda b, r: (b, 0, r, 0)),
            pl.BlockSpec((1, ROWS, 512), lambda b, r: (b, r, 0)),
        ],
        out_specs=pl.BlockSpec((1, 32), lambda b, r: (0, 0),
                               memory_space=pltpu.SMEM),
        out_shape=jax.ShapeDtypeStruct((1, 32), jnp.float32),
        scratch_shapes=[
            pltpu.VMEM((3, NCLS, SUB, 512), jnp.int16),
            pltpu.VMEM((ROWS // SUB, SUB, 512), jnp.int16),
            pltpu.VMEM((ROWS // SUB, SUB, 512), jnp.int16),
        ],
    )(output, target)
    return res[0, :NCLS]
